# baseline (speedup 1.0000x reference)
import functools

import jax
import jax.numpy as jnp
from jax import lax
from jax.experimental import pallas as pl
from jax.experimental.pallas import tpu as pltpu
from jax.experimental.pallas import tpu_sc as plsc

VOCAB = 1000000
D = 64
B = 4096
S = 50
NC = 2
NS = 16
NW = NC * NS
BPW = B // NW
DL = D // 16
SP = 56

_mesh = plsc.VectorSubcoreMesh(core_axis_name="c", subcore_axis_name="s")

PBR = 4000


def _pad_block(x_ref, o_ref):
    o_ref[:, :D] = x_ref[...]
    o_ref[:, D:] = jnp.zeros((PBR, D), jnp.float32)


def _pad_table(table):
    return pl.pallas_call(
        _pad_block,
        grid=(VOCAB // PBR,),
        in_specs=[pl.BlockSpec((PBR, D), lambda i: (i, 0))],
        out_specs=pl.BlockSpec((PBR, 2 * D), lambda i: (i, 0)),
        out_shape=jax.ShapeDtypeStruct((VOCAB, 2 * D), jnp.float32),
    )(table)


@functools.partial(
    pl.kernel,
    mesh=_mesh,
    out_type=jax.ShapeDtypeStruct((NW * SP * BPW,), jnp.int32),
    scratch_types=[
        pltpu.SemaphoreType.DMA,
    ],
    compiler_params=pltpu.CompilerParams(use_tc_tiling_on_sc=True),
)
def _stage_idx(idxt_hbm, out_hbm, sem):
    c = lax.axis_index("c")
    s = lax.axis_index("s")
    w = c * NS + s
    for t in range(SP):
        pltpu.async_copy(idxt_hbm.at[t, pl.ds(BPW * w, BPW)],
                         out_hbm.at[pl.ds(BPW * (SP * w + t), BPW)], sem)
    for t in range(SP):
        pltpu.make_async_copy(idxt_hbm.at[t, pl.ds(BPW * w, BPW)],
                              out_hbm.at[pl.ds(BPW * (SP * w + t), BPW)],
                              sem).wait()


@functools.partial(
    pl.kernel,
    mesh=_mesh,
    out_type=jax.ShapeDtypeStruct((B, D), jnp.float32),
    scratch_types=[
        pltpu.VMEM((SP * BPW,), jnp.int32),
        pltpu.VMEM((BPW, 2 * D), jnp.float32),
        pltpu.VMEM((BPW, 2 * D), jnp.float32),
        pltpu.VMEM((BPW, D), jnp.float32),
        pltpu.SemaphoreType.DMA,
        pltpu.SemaphoreType.DMA,
    ],
    compiler_params=pltpu.CompilerParams(use_tc_tiling_on_sc=True),
)
def _embed_sum(table_hbm, idx_hbm, out_hbm, idx_v, rows_v0, rows_v1,
               out_v, sem0, sem1):
    c = lax.axis_index("c")
    s = lax.axis_index("s")
    w = c * NS + s
    base = w * BPW

    pltpu.sync_copy(idx_hbm.at[pl.ds(SP * BPW * w, SP * BPW)], idx_v)

    bufs = (rows_v0, rows_v1)
    sems = (sem0, sem1)

    def start(t, par):
        pltpu.async_copy(table_hbm.at[idx_v.at[pl.ds(BPW * t, BPW)]],
                         bufs[par], sems[par])

    def wait(t, par):
        pltpu.make_async_copy(table_hbm.at[idx_v.at[pl.ds(BPW * t, BPW)]],
                              bufs[par], sems[par]).wait()

    start(0, 0)
    start(1, 1)
    wait(0, 0)

    def init_body(r, carry):
        for d in range(DL):
            out_v[r, pl.ds(d * 16, 16)] = rows_v0[r, pl.ds(d * 16, 16)]
        return carry
    lax.fori_loop(0, BPW, init_body, 0)

    def acc_from(buf):
        def body(r, carry):
            for d in range(DL):
                x = buf[r, pl.ds(d * 16, 16)]
                y = out_v[r, pl.ds(d * 16, 16)]
                out_v[r, pl.ds(d * 16, 16)] = x + y
            return carry
        lax.fori_loop(0, BPW, body, 0)

    def outer(ss, carry):
        for i in range(2):
            t = 2 * ss + 1 + i
            par = (1 + i) % 2

            @pl.when(t + 1 < S)
            def _():
                start(t + 1, (par + 1) % 2)

            wait(t, par)
            acc_from(bufs[par])
        return carry

    lax.fori_loop(0, (S - 1) // 2, outer, 0)
    wait(S - 1, 1)
    acc_from(bufs[1])

    pltpu.sync_copy(out_v, out_hbm.at[pl.ds(base, BPW)])


def kernel(inputs, table_pri):
    idxt = jnp.transpose(inputs.astype(jnp.int32))
    idxt = jnp.pad(idxt, ((0, SP - S), (0, 0)))
    return _embed_sum(_pad_table(table_pri), _stage_idx(idxt))


# speedup vs baseline: 1.0429x; 1.0429x over previous
import functools

import jax
import jax.numpy as jnp
from jax import lax
from jax.experimental import pallas as pl
from jax.experimental.pallas import tpu as pltpu
from jax.experimental.pallas import tpu_sc as plsc

VOCAB = 1000000
D = 64
B = 4096
S = 50
NC = 2
NS = 16
NW = NC * NS
BPW = B // NW
DL = D // 16
SP = 56

_mesh = plsc.VectorSubcoreMesh(core_axis_name="c", subcore_axis_name="s")

PBR = 8000


def _pad_block(x_ref, o_ref):
    o_ref[:, :D] = x_ref[...]


def _pad_table(table):
    return pl.pallas_call(
        _pad_block,
        grid=(VOCAB // PBR,),
        in_specs=[pl.BlockSpec((PBR, D), lambda i: (i, 0))],
        out_specs=pl.BlockSpec((PBR, 2 * D), lambda i: (i, 0)),
        out_shape=jax.ShapeDtypeStruct((VOCAB, 2 * D), jnp.float32),
    )(table)


@functools.partial(
    pl.kernel,
    mesh=_mesh,
    out_type=jax.ShapeDtypeStruct((NW * SP * BPW,), jnp.int32),
    scratch_types=[
        pltpu.SemaphoreType.DMA,
    ],
    compiler_params=pltpu.CompilerParams(use_tc_tiling_on_sc=True),
)
def _stage_idx(idxt_hbm, out_hbm, sem):
    c = lax.axis_index("c")
    s = lax.axis_index("s")
    w = c * NS + s
    for t in range(SP):
        pltpu.async_copy(idxt_hbm.at[t, pl.ds(BPW * w, BPW)],
                         out_hbm.at[pl.ds(BPW * (SP * w + t), BPW)], sem)
    for t in range(SP):
        pltpu.make_async_copy(idxt_hbm.at[t, pl.ds(BPW * w, BPW)],
                              out_hbm.at[pl.ds(BPW * (SP * w + t), BPW)],
                              sem).wait()


@functools.partial(
    pl.kernel,
    mesh=_mesh,
    out_type=jax.ShapeDtypeStruct((B, D), jnp.float32),
    scratch_types=[
        pltpu.VMEM((SP * BPW,), jnp.int32),
        pltpu.VMEM((BPW, 2 * D), jnp.float32),
        pltpu.VMEM((BPW, 2 * D), jnp.float32),
        pltpu.VMEM((BPW, D), jnp.float32),
        pltpu.SemaphoreType.DMA,
        pltpu.SemaphoreType.DMA,
    ],
    compiler_params=pltpu.CompilerParams(use_tc_tiling_on_sc=True),
)
def _embed_sum(table_hbm, idx_hbm, out_hbm, idx_v, rows_v0, rows_v1,
               out_v, sem0, sem1):
    c = lax.axis_index("c")
    s = lax.axis_index("s")
    w = c * NS + s
    base = w * BPW

    pltpu.sync_copy(idx_hbm.at[pl.ds(SP * BPW * w, SP * BPW)], idx_v)

    bufs = (rows_v0, rows_v1)
    sems = (sem0, sem1)

    def start(t, par):
        pltpu.async_copy(table_hbm.at[idx_v.at[pl.ds(BPW * t, BPW)]],
                         bufs[par], sems[par])

    def wait(t, par):
        pltpu.make_async_copy(table_hbm.at[idx_v.at[pl.ds(BPW * t, BPW)]],
                              bufs[par], sems[par]).wait()

    start(0, 0)
    start(1, 1)
    wait(0, 0)

    def init_body(r, carry):
        for d in range(DL):
            out_v[r, pl.ds(d * 16, 16)] = rows_v0[r, pl.ds(d * 16, 16)]
        return carry
    lax.fori_loop(0, BPW, init_body, 0)

    def acc_from(buf):
        def body(r, carry):
            for d in range(DL):
                x = buf[r, pl.ds(d * 16, 16)]
                y = out_v[r, pl.ds(d * 16, 16)]
                out_v[r, pl.ds(d * 16, 16)] = x + y
            return carry
        lax.fori_loop(0, BPW, body, 0)

    def outer(ss, carry):
        for i in range(2):
            t = 2 * ss + 1 + i
            par = (1 + i) % 2

            @pl.when(t + 1 < S)
            def _():
                start(t + 1, (par + 1) % 2)

            wait(t, par)
            acc_from(bufs[par])
        return carry

    lax.fori_loop(0, (S - 1) // 2, outer, 0)
    wait(S - 1, 1)
    acc_from(bufs[1])

    pltpu.sync_copy(out_v, out_hbm.at[pl.ds(base, BPW)])


def kernel(inputs, table_pri):
    idxt = jnp.transpose(inputs.astype(jnp.int32))
    idxt = jnp.pad(idxt, ((0, SP - S), (0, 0)))
    return _embed_sum(_pad_table(table_pri), _stage_idx(idxt))


# speedup vs baseline: 1.2613x; 1.2095x over previous
import functools

import jax
import jax.numpy as jnp
from jax import lax
from jax.experimental import pallas as pl
from jax.experimental.pallas import tpu as pltpu
from jax.experimental.pallas import tpu_sc as plsc

VOCAB = 1000000
D = 64
B = 4096
S = 50
NC = 2
NS = 16
NW = NC * NS
BPW = B // NW
DL = D // 16
SP = 56

_mesh = plsc.VectorSubcoreMesh(core_axis_name="c", subcore_axis_name="s")

PBR = 8000


def _pad_block(x_ref, o_ref):
    o_ref[:, :D] = x_ref[...]


def _pad_table(table):
    return pl.pallas_call(
        _pad_block,
        grid=(VOCAB // PBR,),
        in_specs=[pl.BlockSpec((PBR, D), lambda i: (i, 0))],
        out_specs=pl.BlockSpec((PBR, 2 * D), lambda i: (i, 0)),
        out_shape=jax.ShapeDtypeStruct((VOCAB, 2 * D), jnp.float32),
    )(table)


@functools.partial(
    pl.kernel,
    mesh=_mesh,
    out_type=jax.ShapeDtypeStruct((NW * SP * BPW,), jnp.int32),
    scratch_types=[
        pltpu.SemaphoreType.DMA,
    ],
    compiler_params=pltpu.CompilerParams(use_tc_tiling_on_sc=True),
)
def _stage_idx(idxt_hbm, out_hbm, sem):
    c = lax.axis_index("c")
    s = lax.axis_index("s")
    w = c * NS + s
    for t in range(SP):
        pltpu.async_copy(idxt_hbm.at[t, pl.ds(BPW * w, BPW)],
                         out_hbm.at[pl.ds(BPW * (SP * w + t), BPW)], sem)
    for t in range(SP):
        pltpu.make_async_copy(idxt_hbm.at[t, pl.ds(BPW * w, BPW)],
                              out_hbm.at[pl.ds(BPW * (SP * w + t), BPW)],
                              sem).wait()


@functools.partial(
    pl.kernel,
    mesh=_mesh,
    out_type=jax.ShapeDtypeStruct((B, D), jnp.float32),
    scratch_types=[
        pltpu.VMEM((SP * BPW,), jnp.int32),
        pltpu.VMEM((BPW, 2 * D), jnp.float32),
        pltpu.VMEM((BPW, 2 * D), jnp.float32),
        pltpu.VMEM((BPW, D), jnp.float32),
        pltpu.SemaphoreType.DMA,
        pltpu.SemaphoreType.DMA,
    ],
    compiler_params=pltpu.CompilerParams(use_tc_tiling_on_sc=True),
)
def _embed_sum(table_hbm, idx_hbm, out_hbm, idx_v, rows_v0, rows_v1,
               out_v, sem0, sem1):
    c = lax.axis_index("c")
    s = lax.axis_index("s")
    w = c * NS + s
    base = w * BPW

    pltpu.sync_copy(idx_hbm.at[pl.ds(SP * BPW * w, SP * BPW)], idx_v)

    bufs = (rows_v0, rows_v1)
    sems = (sem0, sem1)

    def start(t, par):
        pltpu.async_copy(table_hbm.at[idx_v.at[pl.ds(BPW * t, BPW)]],
                         bufs[par], sems[par])

    def wait(t, par):
        pltpu.make_async_copy(table_hbm.at[idx_v.at[pl.ds(BPW * t, BPW)]],
                              bufs[par], sems[par]).wait()

    start(0, 0)
    start(1, 1)
    wait(0, 0)

    def init_body(r, carry):
        for d in range(DL):
            out_v[r, pl.ds(d * 16, 16)] = rows_v0[r, pl.ds(d * 16, 16)]
        return carry
    lax.fori_loop(0, BPW, init_body, 0)

    def acc_from(buf):
        def body(r, carry):
            for d in range(DL):
                x = buf[r, pl.ds(d * 16, 16)]
                y = out_v[r, pl.ds(d * 16, 16)]
                out_v[r, pl.ds(d * 16, 16)] = x + y
            return carry
        lax.fori_loop(0, BPW, body, 0)

    def outer(ss, carry):
        for i in range(2):
            t = 2 * ss + 1 + i
            par = (1 + i) % 2

            @pl.when(t + 1 < S)
            def _():
                start(t + 1, (par + 1) % 2)

            wait(t, par)
            acc_from(bufs[par])
        return carry

    lax.fori_loop(0, (S - 1) // 2, outer, 0)
    wait(S - 1, 1)
    acc_from(bufs[1])

    pltpu.sync_copy(out_v, out_hbm.at[pl.ds(base, BPW)])


def kernel(inputs, table_pri):
    idxt = jnp.transpose(inputs.astype(jnp.int32))
    idxt = jnp.pad(idxt, ((0, SP - S), (0, 0)))
    table_padded = jnp.pad(table_pri, ((0, 0), (0, D)))
    return _embed_sum(table_padded, _stage_idx(idxt))
